# baseline (speedup 1.0000x reference)
import functools
import math

import jax
import jax.numpy as jnp
import numpy as np
from jax import lax
from jax.experimental import pallas as pl
from jax.experimental.pallas import tpu as pltpu
from jax.experimental.pallas import tpu_sc as plsc

_B = 4
_F = 64
_N = 512
_C = 7
_NG = 4
_K = 8
_INF = np.float32(np.inf)

_NW = 32
_RPW = (_B * _N) // _NW


def _dot(a, b, dims):
    return lax.dot_general(a, b, (dims, ((), ())),
                           preferred_element_type=jnp.float32,
                           precision=lax.Precision.HIGHEST)


def _argmax_groups(gref):
    best = gref[0:1, :]
    bg = jnp.zeros((1, _N), jnp.int32)
    for g in range(1, _NG):
        v = gref[g:g + 1, :]
        upd = v > best
        bg = jnp.where(upd, np.int32(g), bg)
        best = jnp.where(upd, v, best)
    return bg


def _cdist(el, al):
    e = jnp.concatenate(el, axis=0)
    a = jnp.concatenate(al, axis=0)
    acc = _dot(e, a, ((0,), (0,)))
    ones_3f = jnp.ones((1, 3 * _F), jnp.float32)
    esq = _dot(e * e, ones_3f, ((0,), (1,)))
    asq = jnp.sum(a * a, axis=0, keepdims=True)
    d2 = esq + asq - 2.0 * acc
    return jnp.sqrt(jnp.maximum(d2, 0.0))


def _normalize_side(planes, m_all):
    ones_1f = jnp.ones((1, _F), jnp.float32)
    ones_1n = jnp.ones((1, _N), jnp.float32)
    cnt = _dot(ones_1n, m_all, ((1,), (1,)))
    s = [_dot(p, m_all, ((1,), (1,))) for p in planes]
    q = [_dot(p * p, m_all, ((1,), (1,))) for p in planes]
    mean = [_dot(ones_1f, sc, ((1,), (0,))) / (_F * cnt) for sc in s]
    mu = sum(s[c] - cnt * mean[c] for c in range(3)) / (3.0 * cnt)
    amc = [mean[c] + mu for c in range(3)]
    var = sum(q[c] - 2.0 * amc[c] * s[c] + cnt * amc[c] * amc[c]
              for c in range(3)) / (3.0 * cnt - 1.0)
    inv = lax.rsqrt(var)
    invp = _dot(inv, m_all, ((1,), (0,)))
    meanp = [_dot(mean[c], m_all, ((1,), (0,))) for c in range(3)]
    return [(planes[c] - meanp[c]) * invp for c in range(3)]


def _tc_body(*refs):
    f32 = jnp.float32
    pe = [refs[c][0] for c in range(3)]
    pa = [refs[3 + c][0] for c in range(3)]
    eg_ref, ag_ref = refs[6], refs[7]
    dg_ref, dn_ref, dv_ref = refs[8:11]

    bg_e = _argmax_groups(eg_ref)
    bg_a = _argmax_groups(ag_ref)
    me_rows = []
    ma_rows = []
    for g in range(_NG):
        me_rows.append((bg_e == g).astype(f32))
        ma_rows.append((bg_a == g).astype(f32))
    me_all = jnp.concatenate(me_rows, axis=0)
    ma_all = jnp.concatenate(ma_rows, axis=0)
    validf = _dot(me_all, ma_all, ((0,), (0,)))
    valid = validf > 0.5

    io_i = lax.broadcasted_iota(jnp.int32, (_F, _F), 0)
    io_j = lax.broadcasted_iota(jnp.int32, (_F, _F), 1)
    shift_m = ((io_j == io_i - 1) | ((io_i == 0) & (io_j == 0))).astype(f32)
    ve = [p - _dot(shift_m, p, ((1,), (0,))) for p in pe]
    va = [p - _dot(shift_m, p, ((1,), (0,))) for p in pa]

    ne = _normalize_side(pe, me_all)
    na = _normalize_side(pa, ma_all)

    dg_ref[0] = jnp.where(valid, _cdist(pe, pa), _INF)
    dn_ref[0] = jnp.where(valid, _cdist(ne, na), _INF)
    dv_ref[0] = _cdist(ve, va)


def _tc_distances(pe, pa, ge, ga):
    spec_p = pl.BlockSpec((1, _F, _N), lambda b: (b, 0, 0))
    spec_g = pl.BlockSpec((_NG, _N), lambda b: (0, 0))
    spec_o = pl.BlockSpec((1, _N, _N), lambda b: (b, 0, 0))
    return pl.pallas_call(
        _tc_body,
        grid=(_B,),
        in_specs=[spec_p] * 6 + [spec_g, spec_g],
        out_specs=[spec_o, spec_o, spec_o],
        out_shape=[jax.ShapeDtypeStruct((_B, _N, _N), jnp.float32)] * 3,
    )(*pe, *pa, ge, ga)


def _perm_body(eg_ref, ag_ref, lo_ref, hi_ref, perm_ref):
    f32 = jnp.float32
    bg_e = _argmax_groups(eg_ref)
    bg_a = _argmax_groups(ag_ref)
    me_rows = [(bg_e == g).astype(f32) for g in range(_NG)]
    ma_rows = [(bg_a == g).astype(f32) for g in range(_NG)]
    ma_all = jnp.concatenate(ma_rows, axis=0)

    nio_i = lax.broadcasted_iota(jnp.int32, (_N, _N), 0)
    nio_j = lax.broadcasted_iota(jnp.int32, (_N, _N), 1)
    eye_n = (nio_i == nio_j).astype(f32)
    upper = (nio_i <= nio_j).astype(f32)
    rank_a = _dot(ma_all, upper, ((1,), (0,)))
    cnt_a = jnp.sum(ma_all, axis=1, keepdims=True)
    starts = []
    acc = jnp.zeros((1, 1), f32)
    for g in range(_NG):
        starts.append(acc)
        acc = acc + cnt_a[g:g + 1]
    pos = sum(ma_rows[g] * (rank_a[g:g + 1] - 1.0 + starts[g])
              for g in range(_NG))
    pos_col = _dot(eye_n, pos, ((1,), (1,))).astype(jnp.int32)
    perm_t = (pos_col == nio_j).astype(f32)
    iota_col_f = _dot(eye_n, lax.broadcasted_iota(
        jnp.int32, (1, _N), 1).astype(f32), ((1,), (1,)))
    ones_16 = jnp.ones((1, 16), f32)
    perm_col = _dot(perm_t, iota_col_f, ((0,), (0,)))
    perm_ref[:, :] = (perm_col * ones_16).astype(jnp.int32)

    lo_row = sum(me_rows[g] * jnp.floor(starts[g] / 16.0) for g in range(_NG))
    hi_row = sum(me_rows[g] * jnp.floor((starts[g] + cnt_a[g:g + 1] + 15.0)
                                        / 16.0)
                 for g in range(_NG))
    lo_ref[:, :] = (_dot(eye_n, lo_row, ((1,), (1,))) * ones_16
                    ).astype(jnp.int32)
    hi_ref[:, :] = (_dot(eye_n, hi_row, ((1,), (1,))) * ones_16
                    ).astype(jnp.int32)


def _perm_bounds(ge, ga):
    spec_g = pl.BlockSpec((_NG, _N), lambda _: (0, 0))
    spec_b = pl.BlockSpec((_N, 16), lambda _: (0, 0))
    return pl.pallas_call(
        _perm_body,
        grid=(1,),
        in_specs=[spec_g, spec_g],
        out_specs=[spec_b, spec_b, spec_b],
        out_shape=[jax.ShapeDtypeStruct((_N, 16), jnp.int32)] * 3,
    )(ge, ga)


def _sc_topk(dg, dn, dv, lo, hi, perm):
    mesh = plsc.VectorSubcoreMesh(core_axis_name="c", subcore_axis_name="s")

    @functools.partial(
        pl.kernel,
        out_type=jax.ShapeDtypeStruct((_NW, 4, 16), jnp.float32),
        mesh=mesh,
        compiler_params=pltpu.CompilerParams(needs_layout_passes=False),
        scratch_types=[
            pltpu.VMEM((_RPW, _N), jnp.float32),
            pltpu.VMEM((_RPW, _N), jnp.float32),
            pltpu.VMEM((_RPW, _N), jnp.float32),
            pltpu.VMEM((_RPW, 16), jnp.int32),
            pltpu.VMEM((_RPW, 16), jnp.int32),
            pltpu.VMEM((_N // 16, 16), jnp.int32),
            pltpu.VMEM((4, 16), jnp.float32),
        ],
    )
    def body(dg_hbm, dn_hbm, dv_hbm, lo_hbm, hi_hbm, perm_hbm, out_hbm,
             dgv, dnv, dvv, lov, hiv, permv, outv):
        wid = lax.axis_index("s") * 2 + lax.axis_index("c")
        base = wid * _RPW
        base_pt = base % _N
        pltpu.sync_copy(dg_hbm.at[pl.ds(base, _RPW)], dgv)
        pltpu.sync_copy(dn_hbm.at[pl.ds(base, _RPW)], dnv)
        pltpu.sync_copy(dv_hbm.at[pl.ds(base, _RPW)], dvv)
        pltpu.sync_copy(lo_hbm.at[pl.ds(base_pt, _RPW)], lov)
        pltpu.sync_copy(hi_hbm.at[pl.ds(base_pt, _RPW)], hiv)
        pltpu.sync_copy(perm_hbm, permv)

        zeros = jnp.zeros((16,), jnp.float32)
        inf16 = jnp.full((16,), _INF, jnp.float32)
        m8 = lax.iota(jnp.int32, 16) < _K

        def row_body(r, acc):
            accg, accn, accv = acc
            lo_s = jnp.max(lov[r, pl.ds(0, 16)])
            hi_s = jnp.max(hiv[r, pl.ds(0, 16)])
            rr = jnp.full((16,), r, jnp.int32)

            def chunk_body(c, st):
                bk, bv, bn = st
                iv = permv[c, pl.ds(0, 16)]
                kc = plsc.load_gather(dgv, [rr, iv])
                vc = plsc.load_gather(dvv, [rr, iv])
                nc = plsc.load_gather(dnv, [rr, iv])
                kd, vd = plsc.sort_key_val(kc, vc, descending=True)
                nd, _ = plsc.sort_key_val(nc, nc, descending=True)
                take = kd < bk
                bk2 = jnp.minimum(bk, kd)
                bv2 = jnp.where(take, vd, bv)
                bn2 = jnp.minimum(bn, nd)
                bk3, bv3 = plsc.sort_key_val(bk2, bv2)
                bn3, _ = plsc.sort_key_val(bn2, bn2)
                return (bk3, bv3, bn3)

            bk, bv, bn = lax.fori_loop(lo_s, hi_s, chunk_body,
                                       (inf16, zeros, inf16))
            accg = accg + jnp.where(m8, bk, zeros)
            accn = accn + jnp.where(m8, bn, zeros)
            accv = accv + jnp.where(m8, bv, zeros)
            return (accg, accn, accv)

        accg, accn, accv = lax.fori_loop(0, _RPW, row_body,
                                         (zeros, zeros, zeros))
        outv[0, :] = accg
        outv[1, :] = accn
        outv[2, :] = accv
        outv[3, :] = zeros
        pltpu.sync_copy(outv, out_hbm.at[wid])

    return body(dg, dn, dv, lo, hi, perm)


def kernel(expected, actual):
    pe = [expected[:, :, :, c] for c in range(3)]
    pa = [actual[:, :, :, c] for c in range(3)]
    ge = jnp.transpose(expected[0, 0, :, 3:])
    ga = jnp.transpose(actual[0, 0, :, 3:])
    lo, hi, perm = _perm_bounds(ge, ga)
    perm32 = perm[:, 0].reshape(_N // 16, 16)
    dg, dn, dv = _tc_distances(pe, pa, ge, ga)
    sums = _sc_topk(dg.reshape(_B * _N, _N),
                    dn.reshape(_B * _N, _N),
                    dv.reshape(_B * _N, _N), lo, hi, perm32)
    tot = jnp.sum(sums, axis=(0, 2))
    denom = np.float32(_B * _N * _K * math.sqrt(_F))
    return (tot[0] / denom, tot[1] / denom, tot[2] / denom)


# speedup vs baseline: 1.0468x; 1.0468x over previous
import functools
import math

import jax
import jax.numpy as jnp
import numpy as np
from jax import lax
from jax.experimental import pallas as pl
from jax.experimental.pallas import tpu as pltpu
from jax.experimental.pallas import tpu_sc as plsc

_B = 4
_F = 64
_N = 512
_C = 7
_NG = 4
_K = 8
_INF = np.float32(np.inf)

_NW = 32
_HB = 2
_RPW = (_HB * _N) // _NW


def _dot(a, b, dims):
    return lax.dot_general(a, b, (dims, ((), ())),
                           preferred_element_type=jnp.float32,
                           precision=lax.Precision.HIGHEST)


def _argmax_groups(gref):
    best = gref[0:1, :]
    bg = jnp.zeros((1, _N), jnp.int32)
    for g in range(1, _NG):
        v = gref[g:g + 1, :]
        upd = v > best
        bg = jnp.where(upd, np.int32(g), bg)
        best = jnp.where(upd, v, best)
    return bg


def _cdist(el, al):
    e = jnp.concatenate(el, axis=0)
    a = jnp.concatenate(al, axis=0)
    acc = _dot(e, a, ((0,), (0,)))
    ones_3f = jnp.ones((1, 3 * _F), jnp.float32)
    esq = _dot(e * e, ones_3f, ((0,), (1,)))
    asq = jnp.sum(a * a, axis=0, keepdims=True)
    d2 = esq + asq - 2.0 * acc
    return jnp.sqrt(jnp.maximum(d2, 0.0))


def _normalize_side(planes, m_all):
    ones_1f = jnp.ones((1, _F), jnp.float32)
    ones_1n = jnp.ones((1, _N), jnp.float32)
    cnt = _dot(ones_1n, m_all, ((1,), (1,)))
    s = [_dot(p, m_all, ((1,), (1,))) for p in planes]
    q = [_dot(p * p, m_all, ((1,), (1,))) for p in planes]
    mean = [_dot(ones_1f, sc, ((1,), (0,))) / (_F * cnt) for sc in s]
    mu = sum(s[c] - cnt * mean[c] for c in range(3)) / (3.0 * cnt)
    amc = [mean[c] + mu for c in range(3)]
    var = sum(q[c] - 2.0 * amc[c] * s[c] + cnt * amc[c] * amc[c]
              for c in range(3)) / (3.0 * cnt - 1.0)
    inv = lax.rsqrt(var)
    invp = _dot(inv, m_all, ((1,), (0,)))
    meanp = [_dot(mean[c], m_all, ((1,), (0,))) for c in range(3)]
    return [(planes[c] - meanp[c]) * invp for c in range(3)]


def _tc_body(*refs):
    f32 = jnp.float32
    pe = [refs[c][0] for c in range(3)]
    pa = [refs[3 + c][0] for c in range(3)]
    eg_ref, ag_ref = refs[6], refs[7]
    dg_ref, dn_ref, dv_ref = refs[8:11]

    bg_e = _argmax_groups(eg_ref)
    bg_a = _argmax_groups(ag_ref)
    me_rows = []
    ma_rows = []
    for g in range(_NG):
        me_rows.append((bg_e == g).astype(f32))
        ma_rows.append((bg_a == g).astype(f32))
    me_all = jnp.concatenate(me_rows, axis=0)
    ma_all = jnp.concatenate(ma_rows, axis=0)
    validf = _dot(me_all, ma_all, ((0,), (0,)))
    valid = validf > 0.5

    io_i = lax.broadcasted_iota(jnp.int32, (_F, _F), 0)
    io_j = lax.broadcasted_iota(jnp.int32, (_F, _F), 1)
    shift_m = ((io_j == io_i - 1) | ((io_i == 0) & (io_j == 0))).astype(f32)
    ve = [p - _dot(shift_m, p, ((1,), (0,))) for p in pe]
    va = [p - _dot(shift_m, p, ((1,), (0,))) for p in pa]

    ne = _normalize_side(pe, me_all)
    na = _normalize_side(pa, ma_all)

    dg_ref[0] = jnp.where(valid, _cdist(pe, pa), _INF)
    dn_ref[0] = jnp.where(valid, _cdist(ne, na), _INF)
    dv_ref[0] = _cdist(ve, va)


def _tc_distances(pe, pa, ge, ga, h):
    spec_p = pl.BlockSpec((1, _F, _N), lambda b: (h * _HB + b, 0, 0))
    spec_g = pl.BlockSpec((_NG, _N), lambda b: (0, 0))
    spec_o = pl.BlockSpec((1, _N, _N), lambda b: (b, 0, 0))
    return pl.pallas_call(
        _tc_body,
        grid=(_HB,),
        in_specs=[spec_p] * 6 + [spec_g, spec_g],
        out_specs=[spec_o, spec_o, spec_o],
        out_shape=[jax.ShapeDtypeStruct((_HB, _N, _N), jnp.float32)] * 3,
    )(*pe, *pa, ge, ga)


def _perm_body(eg_ref, ag_ref, lo_ref, hi_ref, perm_ref):
    f32 = jnp.float32
    bg_e = _argmax_groups(eg_ref)
    bg_a = _argmax_groups(ag_ref)
    me_rows = [(bg_e == g).astype(f32) for g in range(_NG)]
    ma_rows = [(bg_a == g).astype(f32) for g in range(_NG)]
    ma_all = jnp.concatenate(ma_rows, axis=0)

    nio_i = lax.broadcasted_iota(jnp.int32, (_N, _N), 0)
    nio_j = lax.broadcasted_iota(jnp.int32, (_N, _N), 1)
    eye_n = (nio_i == nio_j).astype(f32)
    upper = (nio_i <= nio_j).astype(f32)
    rank_a = _dot(ma_all, upper, ((1,), (0,)))
    cnt_a = jnp.sum(ma_all, axis=1, keepdims=True)
    starts = []
    acc = jnp.zeros((1, 1), f32)
    for g in range(_NG):
        starts.append(acc)
        acc = acc + cnt_a[g:g + 1]
    pos = sum(ma_rows[g] * (rank_a[g:g + 1] - 1.0 + starts[g])
              for g in range(_NG))
    pos_col = _dot(eye_n, pos, ((1,), (1,))).astype(jnp.int32)
    perm_t = (pos_col == nio_j).astype(f32)
    iota_col_f = _dot(eye_n, lax.broadcasted_iota(
        jnp.int32, (1, _N), 1).astype(f32), ((1,), (1,)))
    ones_16 = jnp.ones((1, 16), f32)
    perm_col = _dot(perm_t, iota_col_f, ((0,), (0,)))
    perm_ref[:, :] = (perm_col * ones_16).astype(jnp.int32)

    lo_row = sum(me_rows[g] * jnp.floor(starts[g] / 16.0) for g in range(_NG))
    hi_row = sum(me_rows[g] * jnp.floor((starts[g] + cnt_a[g:g + 1] + 15.0)
                                        / 16.0)
                 for g in range(_NG))
    lo_ref[:, :] = (_dot(eye_n, lo_row, ((1,), (1,))) * ones_16
                    ).astype(jnp.int32)
    hi_ref[:, :] = (_dot(eye_n, hi_row, ((1,), (1,))) * ones_16
                    ).astype(jnp.int32)


def _perm_bounds(ge, ga):
    spec_g = pl.BlockSpec((_NG, _N), lambda _: (0, 0))
    spec_b = pl.BlockSpec((_N, 16), lambda _: (0, 0))
    return pl.pallas_call(
        _perm_body,
        grid=(1,),
        in_specs=[spec_g, spec_g],
        out_specs=[spec_b, spec_b, spec_b],
        out_shape=[jax.ShapeDtypeStruct((_N, 16), jnp.int32)] * 3,
    )(ge, ga)


def _sc_topk(dg, dn, dv, lo, hi, perm):
    mesh = plsc.VectorSubcoreMesh(core_axis_name="c", subcore_axis_name="s")

    @functools.partial(
        pl.kernel,
        out_type=jax.ShapeDtypeStruct((_NW, 4, 16), jnp.float32),
        mesh=mesh,
        compiler_params=pltpu.CompilerParams(needs_layout_passes=False),
        scratch_types=[
            pltpu.VMEM((_RPW, _N), jnp.float32),
            pltpu.VMEM((_RPW, _N), jnp.float32),
            pltpu.VMEM((_RPW, _N), jnp.float32),
            pltpu.VMEM((_RPW, 16), jnp.int32),
            pltpu.VMEM((_RPW, 16), jnp.int32),
            pltpu.VMEM((_N // 16, 16), jnp.int32),
            pltpu.VMEM((4, 16), jnp.float32),
        ],
    )
    def body(dg_hbm, dn_hbm, dv_hbm, lo_hbm, hi_hbm, perm_hbm, out_hbm,
             dgv, dnv, dvv, lov, hiv, permv, outv):
        wid = lax.axis_index("s") * 2 + lax.axis_index("c")
        base = wid * _RPW
        base_pt = base % _N
        pltpu.sync_copy(dg_hbm.at[pl.ds(base, _RPW)], dgv)
        pltpu.sync_copy(dn_hbm.at[pl.ds(base, _RPW)], dnv)
        pltpu.sync_copy(dv_hbm.at[pl.ds(base, _RPW)], dvv)
        pltpu.sync_copy(lo_hbm.at[pl.ds(base_pt, _RPW)], lov)
        pltpu.sync_copy(hi_hbm.at[pl.ds(base_pt, _RPW)], hiv)
        pltpu.sync_copy(perm_hbm, permv)

        zeros = jnp.zeros((16,), jnp.float32)
        inf16 = jnp.full((16,), _INF, jnp.float32)
        m8 = lax.iota(jnp.int32, 16) < _K

        def row_body(r, acc):
            accg, accn, accv = acc
            lo_s = jnp.max(lov[r, pl.ds(0, 16)])
            hi_s = jnp.max(hiv[r, pl.ds(0, 16)])
            rr = jnp.full((16,), r, jnp.int32)

            def chunk_body(c, st):
                bk, bv, bn = st
                iv = permv[c, pl.ds(0, 16)]
                kc = plsc.load_gather(dgv, [rr, iv])
                vc = plsc.load_gather(dvv, [rr, iv])
                nc = plsc.load_gather(dnv, [rr, iv])
                kd, vd = plsc.sort_key_val(kc, vc, descending=True)
                nd, _ = plsc.sort_key_val(nc, nc, descending=True)
                take = kd < bk
                bk2 = jnp.minimum(bk, kd)
                bv2 = jnp.where(take, vd, bv)
                bn2 = jnp.minimum(bn, nd)
                bk3, bv3 = plsc.sort_key_val(bk2, bv2)
                bn3, _ = plsc.sort_key_val(bn2, bn2)
                return (bk3, bv3, bn3)

            bk, bv, bn = lax.fori_loop(lo_s, hi_s, chunk_body,
                                       (inf16, zeros, inf16))
            accg = accg + jnp.where(m8, bk, zeros)
            accn = accn + jnp.where(m8, bn, zeros)
            accv = accv + jnp.where(m8, bv, zeros)
            return (accg, accn, accv)

        accg, accn, accv = lax.fori_loop(0, _RPW, row_body,
                                         (zeros, zeros, zeros))
        outv[0, :] = accg
        outv[1, :] = accn
        outv[2, :] = accv
        outv[3, :] = zeros
        pltpu.sync_copy(outv, out_hbm.at[wid])

    return body(dg, dn, dv, lo, hi, perm)


def kernel(expected, actual):
    pe = [expected[:, :, :, c] for c in range(3)]
    pa = [actual[:, :, :, c] for c in range(3)]
    ge = jnp.transpose(expected[0, 0, :, 3:])
    ga = jnp.transpose(actual[0, 0, :, 3:])
    lo, hi, perm = _perm_bounds(ge, ga)
    perm32 = perm[:, 0].reshape(_N // 16, 16)
    parts = []
    for h in range(_B // _HB):
        dgh, dnh, dvh = _tc_distances(pe, pa, ge, ga, h)
        parts.append(_sc_topk(dgh.reshape(_HB * _N, _N),
                              dnh.reshape(_HB * _N, _N),
                              dvh.reshape(_HB * _N, _N), lo, hi, perm32))
    tot = jnp.sum(jnp.stack(parts), axis=(0, 1, 3))
    denom = np.float32(_B * _N * _K * math.sqrt(_F))
    return (tot[0] / denom, tot[1] / denom, tot[2] / denom)


# speedup vs baseline: 1.0531x; 1.0061x over previous
import functools
import math

import jax
import jax.numpy as jnp
import numpy as np
from jax import lax
from jax.experimental import pallas as pl
from jax.experimental.pallas import tpu as pltpu
from jax.experimental.pallas import tpu_sc as plsc

_B = 4
_F = 64
_N = 512
_C = 7
_NG = 4
_K = 8
_INF = np.float32(np.inf)

_NW = 32
_HB = 2
_RPW = (_HB * _N) // _NW


def _dot(a, b, dims):
    return lax.dot_general(a, b, (dims, ((), ())),
                           preferred_element_type=jnp.float32,
                           precision=lax.Precision.HIGHEST)


def _argmax_groups(gref):
    best = gref[0:1, :]
    bg = jnp.zeros((1, _N), jnp.int32)
    for g in range(1, _NG):
        v = gref[g:g + 1, :]
        upd = v > best
        bg = jnp.where(upd, np.int32(g), bg)
        best = jnp.where(upd, v, best)
    return bg


def _cdist(el, al):
    e = jnp.concatenate(el, axis=0)
    a = jnp.concatenate(al, axis=0)
    acc = _dot(e, a, ((0,), (0,)))
    ones_3f = jnp.ones((1, 3 * _F), jnp.float32)
    esq = _dot(e * e, ones_3f, ((0,), (1,)))
    asq = jnp.sum(a * a, axis=0, keepdims=True)
    d2 = esq + asq - 2.0 * acc
    return jnp.sqrt(jnp.maximum(d2, 0.0))


def _normalize_side(planes, m_all):
    ones_1f = jnp.ones((1, _F), jnp.float32)
    ones_1n = jnp.ones((1, _N), jnp.float32)
    cnt = _dot(ones_1n, m_all, ((1,), (1,)))
    s = [_dot(p, m_all, ((1,), (1,))) for p in planes]
    q = [_dot(p * p, m_all, ((1,), (1,))) for p in planes]
    mean = [_dot(ones_1f, sc, ((1,), (0,))) / (_F * cnt) for sc in s]
    mu = sum(s[c] - cnt * mean[c] for c in range(3)) / (3.0 * cnt)
    amc = [mean[c] + mu for c in range(3)]
    var = sum(q[c] - 2.0 * amc[c] * s[c] + cnt * amc[c] * amc[c]
              for c in range(3)) / (3.0 * cnt - 1.0)
    inv = lax.rsqrt(var)
    invp = _dot(inv, m_all, ((1,), (0,)))
    meanp = [_dot(mean[c], m_all, ((1,), (0,))) for c in range(3)]
    return [(planes[c] - meanp[c]) * invp for c in range(3)]


def _tc_body(*refs):
    f32 = jnp.float32
    pe = [refs[c][0] for c in range(3)]
    pa = [refs[3 + c][0] for c in range(3)]
    eg_ref, ag_ref = refs[6], refs[7]
    dg_ref, dn_ref, dv_ref = refs[8:11]

    bg_e = _argmax_groups(eg_ref)
    bg_a = _argmax_groups(ag_ref)
    me_rows = []
    ma_rows = []
    for g in range(_NG):
        me_rows.append((bg_e == g).astype(f32))
        ma_rows.append((bg_a == g).astype(f32))
    me_all = jnp.concatenate(me_rows, axis=0)
    ma_all = jnp.concatenate(ma_rows, axis=0)
    validf = _dot(me_all, ma_all, ((0,), (0,)))
    valid = validf > 0.5

    io_i = lax.broadcasted_iota(jnp.int32, (_F, _F), 0)
    io_j = lax.broadcasted_iota(jnp.int32, (_F, _F), 1)
    shift_m = ((io_j == io_i - 1) | ((io_i == 0) & (io_j == 0))).astype(f32)
    ve = [p - _dot(shift_m, p, ((1,), (0,))) for p in pe]
    va = [p - _dot(shift_m, p, ((1,), (0,))) for p in pa]

    ne = _normalize_side(pe, me_all)
    na = _normalize_side(pa, ma_all)

    dg_ref[0] = jnp.where(valid, _cdist(pe, pa), _INF)
    dn_ref[0] = jnp.where(valid, _cdist(ne, na), _INF)
    dv_ref[0] = _cdist(ve, va)


def _tc_distances(pe, pa, ge, ga, h):
    spec_p = pl.BlockSpec((1, _F, _N), lambda b: (h * _HB + b, 0, 0))
    spec_g = pl.BlockSpec((_NG, _N), lambda b: (0, 0))
    spec_o = pl.BlockSpec((1, _N, _N), lambda b: (b, 0, 0))
    return pl.pallas_call(
        _tc_body,
        grid=(_HB,),
        in_specs=[spec_p] * 6 + [spec_g, spec_g],
        out_specs=[spec_o, spec_o, spec_o],
        out_shape=[jax.ShapeDtypeStruct((_HB, _N, _N), jnp.float32)] * 3,
    )(*pe, *pa, ge, ga)


def _perm_body(eg_ref, ag_ref, lo_ref, hi_ref, perm_ref):
    f32 = jnp.float32
    bg_e = _argmax_groups(eg_ref)
    bg_a = _argmax_groups(ag_ref)
    me_rows = [(bg_e == g).astype(f32) for g in range(_NG)]
    ma_rows = [(bg_a == g).astype(f32) for g in range(_NG)]
    ma_all = jnp.concatenate(ma_rows, axis=0)

    nio_i = lax.broadcasted_iota(jnp.int32, (_N, _N), 0)
    nio_j = lax.broadcasted_iota(jnp.int32, (_N, _N), 1)
    eye_n = (nio_i == nio_j).astype(f32)
    upper = (nio_i <= nio_j).astype(f32)
    rank_a = _dot(ma_all, upper, ((1,), (0,)))
    cnt_a = jnp.sum(ma_all, axis=1, keepdims=True)
    starts = []
    acc = jnp.zeros((1, 1), f32)
    for g in range(_NG):
        starts.append(acc)
        acc = acc + cnt_a[g:g + 1]
    pos = sum(ma_rows[g] * (rank_a[g:g + 1] - 1.0 + starts[g])
              for g in range(_NG))
    pos_col = _dot(eye_n, pos, ((1,), (1,))).astype(jnp.int32)
    perm_t = (pos_col == nio_j).astype(f32)
    iota_col_f = _dot(eye_n, lax.broadcasted_iota(
        jnp.int32, (1, _N), 1).astype(f32), ((1,), (1,)))
    ones_16 = jnp.ones((1, 16), f32)
    perm_col = _dot(perm_t, iota_col_f, ((0,), (0,)))
    perm_ref[:, :] = (perm_col * ones_16).astype(jnp.int32)

    lo_row = sum(me_rows[g] * jnp.floor(starts[g] / 16.0) for g in range(_NG))
    hi_row = sum(me_rows[g] * jnp.floor((starts[g] + cnt_a[g:g + 1] + 15.0)
                                        / 16.0)
                 for g in range(_NG))
    lo_ref[:, :] = (_dot(eye_n, lo_row, ((1,), (1,))) * ones_16
                    ).astype(jnp.int32)
    hi_ref[:, :] = (_dot(eye_n, hi_row, ((1,), (1,))) * ones_16
                    ).astype(jnp.int32)


def _tc_body_with_perm(*refs):
    _tc_body(*refs[:11])

    @pl.when(pl.program_id(0) == 0)
    def _():
        _perm_body(refs[6], refs[7], *refs[11:14])


def _tc_distances_with_perm(pe, pa, ge, ga):
    spec_p = pl.BlockSpec((1, _F, _N), lambda b: (b, 0, 0))
    spec_g = pl.BlockSpec((_NG, _N), lambda b: (0, 0))
    spec_o = pl.BlockSpec((1, _N, _N), lambda b: (b, 0, 0))
    spec_b = pl.BlockSpec((_N, 16), lambda b: (0, 0))
    return pl.pallas_call(
        _tc_body_with_perm,
        grid=(_HB,),
        in_specs=[spec_p] * 6 + [spec_g, spec_g],
        out_specs=[spec_o, spec_o, spec_o, spec_b, spec_b, spec_b],
        out_shape=[jax.ShapeDtypeStruct((_HB, _N, _N), jnp.float32)] * 3
        + [jax.ShapeDtypeStruct((_N, 16), jnp.int32)] * 3,
    )(*pe, *pa, ge, ga)


def _sc_topk(dg, dn, dv, lo, hi, perm):
    mesh = plsc.VectorSubcoreMesh(core_axis_name="c", subcore_axis_name="s")

    @functools.partial(
        pl.kernel,
        out_type=jax.ShapeDtypeStruct((_NW, 4, 16), jnp.float32),
        mesh=mesh,
        compiler_params=pltpu.CompilerParams(needs_layout_passes=False),
        scratch_types=[
            pltpu.VMEM((_RPW, _N), jnp.float32),
            pltpu.VMEM((_RPW, _N), jnp.float32),
            pltpu.VMEM((_RPW, _N), jnp.float32),
            pltpu.VMEM((_RPW, 16), jnp.int32),
            pltpu.VMEM((_RPW, 16), jnp.int32),
            pltpu.VMEM((_N // 16, 16), jnp.int32),
            pltpu.VMEM((4, 16), jnp.float32),
        ],
    )
    def body(dg_hbm, dn_hbm, dv_hbm, lo_hbm, hi_hbm, perm_hbm, out_hbm,
             dgv, dnv, dvv, lov, hiv, permv, outv):
        wid = lax.axis_index("s") * 2 + lax.axis_index("c")
        base = wid * _RPW
        base_pt = base % _N
        pltpu.sync_copy(dg_hbm.at[pl.ds(base, _RPW)], dgv)
        pltpu.sync_copy(dn_hbm.at[pl.ds(base, _RPW)], dnv)
        pltpu.sync_copy(dv_hbm.at[pl.ds(base, _RPW)], dvv)
        pltpu.sync_copy(lo_hbm.at[pl.ds(base_pt, _RPW)], lov)
        pltpu.sync_copy(hi_hbm.at[pl.ds(base_pt, _RPW)], hiv)
        pltpu.sync_copy(perm_hbm, permv)

        zeros = jnp.zeros((16,), jnp.float32)
        inf16 = jnp.full((16,), _INF, jnp.float32)
        m8 = lax.iota(jnp.int32, 16) < _K

        def row_body(r, acc):
            accg, accn, accv = acc
            lo_s = jnp.max(lov[r, pl.ds(0, 16)])
            hi_s = jnp.max(hiv[r, pl.ds(0, 16)])
            rr = jnp.full((16,), r, jnp.int32)

            def chunk_body(c, st):
                bk, bv, bn = st
                iv = permv[c, pl.ds(0, 16)]
                kc = plsc.load_gather(dgv, [rr, iv])
                vc = plsc.load_gather(dvv, [rr, iv])
                nc = plsc.load_gather(dnv, [rr, iv])
                kd, vd = plsc.sort_key_val(kc, vc, descending=True)
                nd, _ = plsc.sort_key_val(nc, nc, descending=True)
                take = kd < bk
                bk2 = jnp.minimum(bk, kd)
                bv2 = jnp.where(take, vd, bv)
                bn2 = jnp.minimum(bn, nd)
                bk3, bv3 = plsc.sort_key_val(bk2, bv2)
                bn3, _ = plsc.sort_key_val(bn2, bn2)
                return (bk3, bv3, bn3)

            bk, bv, bn = lax.fori_loop(lo_s, hi_s, chunk_body,
                                       (inf16, zeros, inf16))
            accg = accg + jnp.where(m8, bk, zeros)
            accn = accn + jnp.where(m8, bn, zeros)
            accv = accv + jnp.where(m8, bv, zeros)
            return (accg, accn, accv)

        accg, accn, accv = lax.fori_loop(0, _RPW, row_body,
                                         (zeros, zeros, zeros))
        outv[0, :] = accg
        outv[1, :] = accn
        outv[2, :] = accv
        outv[3, :] = zeros
        pltpu.sync_copy(outv, out_hbm.at[wid])

    return body(dg, dn, dv, lo, hi, perm)


def kernel(expected, actual):
    pe = [expected[:, :, :, c] for c in range(3)]
    pa = [actual[:, :, :, c] for c in range(3)]
    ge = jnp.transpose(expected[0, 0, :, 3:])
    ga = jnp.transpose(actual[0, 0, :, 3:])
    parts = []
    for h in range(_B // _HB):
        if h == 0:
            dgh, dnh, dvh, lo, hi, perm = _tc_distances_with_perm(
                pe, pa, ge, ga)
            perm32 = perm[:, 0].reshape(_N // 16, 16)
        else:
            dgh, dnh, dvh = _tc_distances(pe, pa, ge, ga, h)
        parts.append(_sc_topk(dgh.reshape(_HB * _N, _N),
                              dnh.reshape(_HB * _N, _N),
                              dvh.reshape(_HB * _N, _N), lo, hi, perm32))
    tot = jnp.sum(jnp.stack(parts), axis=(0, 1, 3))
    denom = np.float32(_B * _N * _K * math.sqrt(_F))
    return (tot[0] / denom, tot[1] / denom, tot[2] / denom)


# speedup vs baseline: 1.1143x; 1.0580x over previous
import functools
import math

import jax
import jax.numpy as jnp
import numpy as np
from jax import lax
from jax.experimental import pallas as pl
from jax.experimental.pallas import tpu as pltpu
from jax.experimental.pallas import tpu_sc as plsc

_B = 4
_F = 64
_N = 512
_C = 7
_NG = 4
_K = 8
_INF = np.float32(np.inf)

_NW = 32
_HB = 2
_RPW = (_HB * _N) // _NW


def _dot(a, b, dims):
    return lax.dot_general(a, b, (dims, ((), ())),
                           preferred_element_type=jnp.float32,
                           precision=lax.Precision.HIGHEST)


def _argmax_groups(gref):
    best = gref[0:1, :]
    bg = jnp.zeros((1, _N), jnp.int32)
    for g in range(1, _NG):
        v = gref[g:g + 1, :]
        upd = v > best
        bg = jnp.where(upd, np.int32(g), bg)
        best = jnp.where(upd, v, best)
    return bg


def _dot3x(a, b):
    f32, bf = jnp.float32, jnp.bfloat16
    ah = a.astype(bf)
    al = (a - ah.astype(f32)).astype(bf)
    bh = b.astype(bf)
    bl = (b - bh.astype(f32)).astype(bf)

    def d(x, y):
        return lax.dot_general(x, y, ((((0,), (0,))), ((), ())),
                               preferred_element_type=f32)

    return d(ah, bh) + d(ah, bl) + d(al, bh)


def _cdist(el, al):
    e = jnp.concatenate(el, axis=0)
    a = jnp.concatenate(al, axis=0)
    acc = _dot3x(e, a)
    ones_3f = jnp.ones((1, 3 * _F), jnp.float32)
    esq = _dot(e * e, ones_3f, ((0,), (1,)))
    asq = jnp.sum(a * a, axis=0, keepdims=True)
    d2 = esq + asq - 2.0 * acc
    return jnp.sqrt(jnp.maximum(d2, 0.0))


def _normalize_side(planes, m_all):
    ones_1f = jnp.ones((1, _F), jnp.float32)
    ones_1n = jnp.ones((1, _N), jnp.float32)
    cnt = _dot(ones_1n, m_all, ((1,), (1,)))
    s = [_dot(p, m_all, ((1,), (1,))) for p in planes]
    q = [_dot(p * p, m_all, ((1,), (1,))) for p in planes]
    mean = [_dot(ones_1f, sc, ((1,), (0,))) / (_F * cnt) for sc in s]
    mu = sum(s[c] - cnt * mean[c] for c in range(3)) / (3.0 * cnt)
    amc = [mean[c] + mu for c in range(3)]
    var = sum(q[c] - 2.0 * amc[c] * s[c] + cnt * amc[c] * amc[c]
              for c in range(3)) / (3.0 * cnt - 1.0)
    inv = lax.rsqrt(var)
    invp = _dot(inv, m_all, ((1,), (0,)))
    meanp = [_dot(mean[c], m_all, ((1,), (0,))) for c in range(3)]
    return [(planes[c] - meanp[c]) * invp for c in range(3)]


def _tc_body(*refs):
    f32 = jnp.float32
    pe = [refs[c][0] for c in range(3)]
    pa = [refs[3 + c][0] for c in range(3)]
    eg_ref, ag_ref = refs[6], refs[7]
    dg_ref, dn_ref, dv_ref = refs[8:11]

    bg_e = _argmax_groups(eg_ref)
    bg_a = _argmax_groups(ag_ref)
    me_rows = []
    ma_rows = []
    for g in range(_NG):
        me_rows.append((bg_e == g).astype(f32))
        ma_rows.append((bg_a == g).astype(f32))
    me_all = jnp.concatenate(me_rows, axis=0)
    ma_all = jnp.concatenate(ma_rows, axis=0)
    validf = _dot(me_all, ma_all, ((0,), (0,)))
    valid = validf > 0.5

    io_i = lax.broadcasted_iota(jnp.int32, (_F, _F), 0)
    io_j = lax.broadcasted_iota(jnp.int32, (_F, _F), 1)
    shift_m = ((io_j == io_i - 1) | ((io_i == 0) & (io_j == 0))).astype(f32)
    ve = [p - _dot(shift_m, p, ((1,), (0,))) for p in pe]
    va = [p - _dot(shift_m, p, ((1,), (0,))) for p in pa]

    ne = _normalize_side(pe, me_all)
    na = _normalize_side(pa, ma_all)

    dg_ref[0] = jnp.where(valid, _cdist(pe, pa), _INF)
    dn_ref[0] = jnp.where(valid, _cdist(ne, na), _INF)
    dv_ref[0] = _cdist(ve, va)


def _tc_distances(pe, pa, ge, ga, h):
    spec_p = pl.BlockSpec((1, _F, _N), lambda b: (h * _HB + b, 0, 0))
    spec_g = pl.BlockSpec((_NG, _N), lambda b: (0, 0))
    spec_o = pl.BlockSpec((1, _N, _N), lambda b: (b, 0, 0))
    return pl.pallas_call(
        _tc_body,
        grid=(_HB,),
        in_specs=[spec_p] * 6 + [spec_g, spec_g],
        out_specs=[spec_o, spec_o, spec_o],
        out_shape=[jax.ShapeDtypeStruct((_HB, _N, _N), jnp.float32)] * 3,
    )(*pe, *pa, ge, ga)


def _perm_body(eg_ref, ag_ref, lo_ref, hi_ref, perm_ref):
    f32 = jnp.float32
    bg_e = _argmax_groups(eg_ref)
    bg_a = _argmax_groups(ag_ref)
    me_rows = [(bg_e == g).astype(f32) for g in range(_NG)]
    ma_rows = [(bg_a == g).astype(f32) for g in range(_NG)]
    ma_all = jnp.concatenate(ma_rows, axis=0)

    nio_i = lax.broadcasted_iota(jnp.int32, (_N, _N), 0)
    nio_j = lax.broadcasted_iota(jnp.int32, (_N, _N), 1)
    eye_n = (nio_i == nio_j).astype(f32)
    upper = (nio_i <= nio_j).astype(f32)
    rank_a = _dot(ma_all, upper, ((1,), (0,)))
    cnt_a = jnp.sum(ma_all, axis=1, keepdims=True)
    starts = []
    acc = jnp.zeros((1, 1), f32)
    for g in range(_NG):
        starts.append(acc)
        acc = acc + cnt_a[g:g + 1]
    pos = sum(ma_rows[g] * (rank_a[g:g + 1] - 1.0 + starts[g])
              for g in range(_NG))
    pos_col = _dot(eye_n, pos, ((1,), (1,))).astype(jnp.int32)
    perm_t = (pos_col == nio_j).astype(f32)
    iota_col_f = _dot(eye_n, lax.broadcasted_iota(
        jnp.int32, (1, _N), 1).astype(f32), ((1,), (1,)))
    ones_16 = jnp.ones((1, 16), f32)
    perm_col = _dot(perm_t, iota_col_f, ((0,), (0,)))
    perm_ref[:, :] = (perm_col * ones_16).astype(jnp.int32)

    lo_row = sum(me_rows[g] * jnp.floor(starts[g] / 16.0) for g in range(_NG))
    hi_row = sum(me_rows[g] * jnp.floor((starts[g] + cnt_a[g:g + 1] + 15.0)
                                        / 16.0)
                 for g in range(_NG))
    lo_ref[:, :] = (_dot(eye_n, lo_row, ((1,), (1,))) * ones_16
                    ).astype(jnp.int32)
    hi_ref[:, :] = (_dot(eye_n, hi_row, ((1,), (1,))) * ones_16
                    ).astype(jnp.int32)


def _tc_body_with_perm(*refs):
    _tc_body(*refs[:11])

    @pl.when(pl.program_id(0) == 0)
    def _():
        _perm_body(refs[6], refs[7], *refs[11:14])


def _tc_distances_with_perm(pe, pa, ge, ga):
    spec_p = pl.BlockSpec((1, _F, _N), lambda b: (b, 0, 0))
    spec_g = pl.BlockSpec((_NG, _N), lambda b: (0, 0))
    spec_o = pl.BlockSpec((1, _N, _N), lambda b: (b, 0, 0))
    spec_b = pl.BlockSpec((_N, 16), lambda b: (0, 0))
    return pl.pallas_call(
        _tc_body_with_perm,
        grid=(_HB,),
        in_specs=[spec_p] * 6 + [spec_g, spec_g],
        out_specs=[spec_o, spec_o, spec_o, spec_b, spec_b, spec_b],
        out_shape=[jax.ShapeDtypeStruct((_HB, _N, _N), jnp.float32)] * 3
        + [jax.ShapeDtypeStruct((_N, 16), jnp.int32)] * 3,
    )(*pe, *pa, ge, ga)


def _sc_topk(dg, dn, dv, lo, hi, perm):
    mesh = plsc.VectorSubcoreMesh(core_axis_name="c", subcore_axis_name="s")

    @functools.partial(
        pl.kernel,
        out_type=jax.ShapeDtypeStruct((_NW, 4, 16), jnp.float32),
        mesh=mesh,
        compiler_params=pltpu.CompilerParams(needs_layout_passes=False),
        scratch_types=[
            pltpu.VMEM((_RPW, _N), jnp.float32),
            pltpu.VMEM((_RPW, _N), jnp.float32),
            pltpu.VMEM((_RPW, _N), jnp.float32),
            pltpu.VMEM((_RPW, 16), jnp.int32),
            pltpu.VMEM((_RPW, 16), jnp.int32),
            pltpu.VMEM((_N // 16, 16), jnp.int32),
            pltpu.VMEM((4, 16), jnp.float32),
        ],
    )
    def body(dg_hbm, dn_hbm, dv_hbm, lo_hbm, hi_hbm, perm_hbm, out_hbm,
             dgv, dnv, dvv, lov, hiv, permv, outv):
        wid = lax.axis_index("s") * 2 + lax.axis_index("c")
        base = wid * _RPW
        base_pt = base % _N
        pltpu.sync_copy(dg_hbm.at[pl.ds(base, _RPW)], dgv)
        pltpu.sync_copy(dn_hbm.at[pl.ds(base, _RPW)], dnv)
        pltpu.sync_copy(dv_hbm.at[pl.ds(base, _RPW)], dvv)
        pltpu.sync_copy(lo_hbm.at[pl.ds(base_pt, _RPW)], lov)
        pltpu.sync_copy(hi_hbm.at[pl.ds(base_pt, _RPW)], hiv)
        pltpu.sync_copy(perm_hbm, permv)

        zeros = jnp.zeros((16,), jnp.float32)
        inf16 = jnp.full((16,), _INF, jnp.float32)
        m8 = lax.iota(jnp.int32, 16) < _K

        def row_body(r, acc):
            accg, accn, accv = acc
            lo_s = jnp.max(lov[r, pl.ds(0, 16)])
            hi_s = jnp.max(hiv[r, pl.ds(0, 16)])
            rr = jnp.full((16,), r, jnp.int32)

            def chunk_body(c, st):
                bk, bv, bn = st
                iv = permv[c, pl.ds(0, 16)]
                kc = plsc.load_gather(dgv, [rr, iv])
                vc = plsc.load_gather(dvv, [rr, iv])
                nc = plsc.load_gather(dnv, [rr, iv])
                kd, vd = plsc.sort_key_val(kc, vc, descending=True)
                nd, _ = plsc.sort_key_val(nc, nc, descending=True)
                take = kd < bk
                bk2 = jnp.minimum(bk, kd)
                bv2 = jnp.where(take, vd, bv)
                bn2 = jnp.minimum(bn, nd)
                bk3, bv3 = plsc.sort_key_val(bk2, bv2)
                bn3, _ = plsc.sort_key_val(bn2, bn2)
                return (bk3, bv3, bn3)

            bk, bv, bn = lax.fori_loop(lo_s, hi_s, chunk_body,
                                       (inf16, zeros, inf16))
            accg = accg + jnp.where(m8, bk, zeros)
            accn = accn + jnp.where(m8, bn, zeros)
            accv = accv + jnp.where(m8, bv, zeros)
            return (accg, accn, accv)

        accg, accn, accv = lax.fori_loop(0, _RPW, row_body,
                                         (zeros, zeros, zeros))
        outv[0, :] = accg
        outv[1, :] = accn
        outv[2, :] = accv
        outv[3, :] = zeros
        pltpu.sync_copy(outv, out_hbm.at[wid])

    return body(dg, dn, dv, lo, hi, perm)


def kernel(expected, actual):
    pe = [expected[:, :, :, c] for c in range(3)]
    pa = [actual[:, :, :, c] for c in range(3)]
    ge = jnp.transpose(expected[0, 0, :, 3:])
    ga = jnp.transpose(actual[0, 0, :, 3:])
    parts = []
    for h in range(_B // _HB):
        if h == 0:
            dgh, dnh, dvh, lo, hi, perm = _tc_distances_with_perm(
                pe, pa, ge, ga)
            perm32 = perm[:, 0].reshape(_N // 16, 16)
        else:
            dgh, dnh, dvh = _tc_distances(pe, pa, ge, ga, h)
        parts.append(_sc_topk(dgh.reshape(_HB * _N, _N),
                              dnh.reshape(_HB * _N, _N),
                              dvh.reshape(_HB * _N, _N), lo, hi, perm32))
    tot = jnp.sum(jnp.stack(parts), axis=(0, 1, 3))
    denom = np.float32(_B * _N * _K * math.sqrt(_F))
    return (tot[0] / denom, tot[1] / denom, tot[2] / denom)


# speedup vs baseline: 1.1759x; 1.0553x over previous
import functools
import math

import jax
import jax.numpy as jnp
import numpy as np
from jax import lax
from jax.experimental import pallas as pl
from jax.experimental.pallas import tpu as pltpu
from jax.experimental.pallas import tpu_sc as plsc

_B = 4
_F = 64
_N = 512
_C = 7
_NG = 4
_K = 8
_INF = np.float32(np.inf)

_NW = 32
_HB = 2
_RPW = (_HB * _N) // _NW


def _dot(a, b, dims):
    return lax.dot_general(a, b, (dims, ((), ())),
                           preferred_element_type=jnp.float32,
                           precision=lax.Precision.HIGHEST)


def _argmax_groups(gref):
    best = gref[0:1, :]
    bg = jnp.zeros((1, _N), jnp.int32)
    for g in range(1, _NG):
        v = gref[g:g + 1, :]
        upd = v > best
        bg = jnp.where(upd, np.int32(g), bg)
        best = jnp.where(upd, v, best)
    return bg


def _dot3x(a, b):
    f32, bf = jnp.float32, jnp.bfloat16
    ah = a.astype(bf)
    al = (a - ah.astype(f32)).astype(bf)
    bh = b.astype(bf)
    bl = (b - bh.astype(f32)).astype(bf)

    def d(x, y):
        return lax.dot_general(x, y, ((((0,), (0,))), ((), ())),
                               preferred_element_type=f32)

    return d(ah, bh) + d(ah, bl) + d(al, bh)


def _dot_sel(x, sel, dims):
    f32, bf = jnp.float32, jnp.bfloat16
    xh = x.astype(bf)
    xl = (x - xh.astype(f32)).astype(bf)
    sb = sel.astype(bf)

    def d(u, v):
        return lax.dot_general(u, v, (dims, ((), ())),
                               preferred_element_type=f32)

    return d(xh, sb) + d(xl, sb)


def _dot_sel_l(sel, x, dims):
    f32, bf = jnp.float32, jnp.bfloat16
    xh = x.astype(bf)
    xl = (x - xh.astype(f32)).astype(bf)
    sb = sel.astype(bf)

    def d(u, v):
        return lax.dot_general(u, v, (dims, ((), ())),
                               preferred_element_type=f32)

    return d(sb, xh) + d(sb, xl)


def _cdist(el, al):
    e = jnp.concatenate(el, axis=0)
    a = jnp.concatenate(al, axis=0)
    acc = _dot3x(e, a)
    ones_3f = jnp.ones((1, 3 * _F), jnp.float32)
    esq = _dot(e * e, ones_3f, ((0,), (1,)))
    asq = jnp.sum(a * a, axis=0, keepdims=True)
    d2 = esq + asq - 2.0 * acc
    return jnp.sqrt(jnp.maximum(d2, 0.0))


def _normalize_side(planes, m_all):
    ones_1f = jnp.ones((1, _F), jnp.float32)
    ones_1n = jnp.ones((1, _N), jnp.float32)
    cnt = _dot(ones_1n, m_all, ((1,), (1,)))
    s = [_dot_sel(p, m_all, ((1,), (1,))) for p in planes]
    q = [_dot_sel(p * p, m_all, ((1,), (1,))) for p in planes]
    mean = [_dot(ones_1f, sc, ((1,), (0,))) / (_F * cnt) for sc in s]
    mu = sum(s[c] - cnt * mean[c] for c in range(3)) / (3.0 * cnt)
    amc = [mean[c] + mu for c in range(3)]
    var = sum(q[c] - 2.0 * amc[c] * s[c] + cnt * amc[c] * amc[c]
              for c in range(3)) / (3.0 * cnt - 1.0)
    inv = lax.rsqrt(var)
    invp = _dot_sel(inv, m_all, ((1,), (0,)))
    meanp = [_dot(mean[c], m_all, ((1,), (0,))) for c in range(3)]
    return [(planes[c] - meanp[c]) * invp for c in range(3)]


def _tc_body(*refs):
    f32 = jnp.float32
    pe = [refs[c][0] for c in range(3)]
    pa = [refs[3 + c][0] for c in range(3)]
    eg_ref, ag_ref = refs[6], refs[7]
    dg_ref, dn_ref, dv_ref = refs[8:11]

    bg_e = _argmax_groups(eg_ref)
    bg_a = _argmax_groups(ag_ref)
    me_rows = []
    ma_rows = []
    for g in range(_NG):
        me_rows.append((bg_e == g).astype(f32))
        ma_rows.append((bg_a == g).astype(f32))
    me_all = jnp.concatenate(me_rows, axis=0)
    ma_all = jnp.concatenate(ma_rows, axis=0)
    validf = _dot(me_all, ma_all, ((0,), (0,)))
    valid = validf > 0.5

    io_i = lax.broadcasted_iota(jnp.int32, (_F, _F), 0)
    io_j = lax.broadcasted_iota(jnp.int32, (_F, _F), 1)
    shift_m = ((io_j == io_i - 1) | ((io_i == 0) & (io_j == 0))).astype(f32)
    ve = [p - _dot_sel_l(shift_m, p, ((1,), (0,))) for p in pe]
    va = [p - _dot_sel_l(shift_m, p, ((1,), (0,))) for p in pa]

    ne = _normalize_side(pe, me_all)
    na = _normalize_side(pa, ma_all)

    dg_ref[0] = jnp.where(valid, _cdist(pe, pa), _INF)
    dn_ref[0] = jnp.where(valid, _cdist(ne, na), _INF)
    dv_ref[0] = _cdist(ve, va)


def _tc_distances(pe, pa, ge, ga, h):
    spec_p = pl.BlockSpec((1, _F, _N), lambda b: (h * _HB + b, 0, 0))
    spec_g = pl.BlockSpec((_NG, _N), lambda b: (0, 0))
    spec_o = pl.BlockSpec((1, _N, _N), lambda b: (b, 0, 0))
    return pl.pallas_call(
        _tc_body,
        grid=(_HB,),
        in_specs=[spec_p] * 6 + [spec_g, spec_g],
        out_specs=[spec_o, spec_o, spec_o],
        out_shape=[jax.ShapeDtypeStruct((_HB, _N, _N), jnp.float32)] * 3,
    )(*pe, *pa, ge, ga)


def _perm_body(eg_ref, ag_ref, lo_ref, hi_ref, perm_ref):
    f32 = jnp.float32
    bg_e = _argmax_groups(eg_ref)
    bg_a = _argmax_groups(ag_ref)
    me_rows = [(bg_e == g).astype(f32) for g in range(_NG)]
    ma_rows = [(bg_a == g).astype(f32) for g in range(_NG)]
    ma_all = jnp.concatenate(ma_rows, axis=0)

    nio_i = lax.broadcasted_iota(jnp.int32, (_N, _N), 0)
    nio_j = lax.broadcasted_iota(jnp.int32, (_N, _N), 1)
    eye_n = (nio_i == nio_j).astype(f32)
    upper = (nio_i <= nio_j).astype(f32)
    rank_a = _dot(ma_all, upper, ((1,), (0,)))
    cnt_a = jnp.sum(ma_all, axis=1, keepdims=True)
    starts = []
    acc = jnp.zeros((1, 1), f32)
    for g in range(_NG):
        starts.append(acc)
        acc = acc + cnt_a[g:g + 1]
    pos = sum(ma_rows[g] * (rank_a[g:g + 1] - 1.0 + starts[g])
              for g in range(_NG))
    pos_col = _dot(eye_n, pos, ((1,), (1,))).astype(jnp.int32)
    perm_t = (pos_col == nio_j).astype(f32)
    iota_col_f = _dot(eye_n, lax.broadcasted_iota(
        jnp.int32, (1, _N), 1).astype(f32), ((1,), (1,)))
    ones_16 = jnp.ones((1, 16), f32)
    perm_col = _dot(perm_t, iota_col_f, ((0,), (0,)))
    perm_ref[:, :] = (perm_col * ones_16).astype(jnp.int32)

    lo_row = sum(me_rows[g] * jnp.floor(starts[g] / 16.0) for g in range(_NG))
    hi_row = sum(me_rows[g] * jnp.floor((starts[g] + cnt_a[g:g + 1] + 15.0)
                                        / 16.0)
                 for g in range(_NG))
    lo_ref[:, :] = (_dot(eye_n, lo_row, ((1,), (1,))) * ones_16
                    ).astype(jnp.int32)
    hi_ref[:, :] = (_dot(eye_n, hi_row, ((1,), (1,))) * ones_16
                    ).astype(jnp.int32)


def _tc_body_with_perm(*refs):
    _tc_body(*refs[:11])

    @pl.when(pl.program_id(0) == 0)
    def _():
        _perm_body(refs[6], refs[7], *refs[11:14])


def _tc_distances_with_perm(pe, pa, ge, ga):
    spec_p = pl.BlockSpec((1, _F, _N), lambda b: (b, 0, 0))
    spec_g = pl.BlockSpec((_NG, _N), lambda b: (0, 0))
    spec_o = pl.BlockSpec((1, _N, _N), lambda b: (b, 0, 0))
    spec_b = pl.BlockSpec((_N, 16), lambda b: (0, 0))
    return pl.pallas_call(
        _tc_body_with_perm,
        grid=(_HB,),
        in_specs=[spec_p] * 6 + [spec_g, spec_g],
        out_specs=[spec_o, spec_o, spec_o, spec_b, spec_b, spec_b],
        out_shape=[jax.ShapeDtypeStruct((_HB, _N, _N), jnp.float32)] * 3
        + [jax.ShapeDtypeStruct((_N, 16), jnp.int32)] * 3,
    )(*pe, *pa, ge, ga)


def _sc_topk(dg, dn, dv, lo, hi, perm):
    mesh = plsc.VectorSubcoreMesh(core_axis_name="c", subcore_axis_name="s")

    @functools.partial(
        pl.kernel,
        out_type=jax.ShapeDtypeStruct((_NW, 4, 16), jnp.float32),
        mesh=mesh,
        compiler_params=pltpu.CompilerParams(needs_layout_passes=False),
        scratch_types=[
            pltpu.VMEM((_RPW, _N), jnp.float32),
            pltpu.VMEM((_RPW, _N), jnp.float32),
            pltpu.VMEM((_RPW, _N), jnp.float32),
            pltpu.VMEM((_RPW, 16), jnp.int32),
            pltpu.VMEM((_RPW, 16), jnp.int32),
            pltpu.VMEM((_N // 16, 16), jnp.int32),
            pltpu.VMEM((4, 16), jnp.float32),
        ],
    )
    def body(dg_hbm, dn_hbm, dv_hbm, lo_hbm, hi_hbm, perm_hbm, out_hbm,
             dgv, dnv, dvv, lov, hiv, permv, outv):
        wid = lax.axis_index("s") * 2 + lax.axis_index("c")
        base = wid * _RPW
        base_pt = base % _N
        pltpu.sync_copy(dg_hbm.at[pl.ds(base, _RPW)], dgv)
        pltpu.sync_copy(dn_hbm.at[pl.ds(base, _RPW)], dnv)
        pltpu.sync_copy(dv_hbm.at[pl.ds(base, _RPW)], dvv)
        pltpu.sync_copy(lo_hbm.at[pl.ds(base_pt, _RPW)], lov)
        pltpu.sync_copy(hi_hbm.at[pl.ds(base_pt, _RPW)], hiv)
        pltpu.sync_copy(perm_hbm, permv)

        zeros = jnp.zeros((16,), jnp.float32)
        inf16 = jnp.full((16,), _INF, jnp.float32)
        m8 = lax.iota(jnp.int32, 16) < _K

        def row_body(r, acc):
            accg, accn, accv = acc
            lo_s = jnp.max(lov[r, pl.ds(0, 16)])
            hi_s = jnp.max(hiv[r, pl.ds(0, 16)])
            rr = jnp.full((16,), r, jnp.int32)

            def chunk_body(c, st):
                bk, bv, bn = st
                iv = permv[c, pl.ds(0, 16)]
                kc = plsc.load_gather(dgv, [rr, iv])
                vc = plsc.load_gather(dvv, [rr, iv])
                nc = plsc.load_gather(dnv, [rr, iv])
                kd, vd = plsc.sort_key_val(kc, vc, descending=True)
                nd, _ = plsc.sort_key_val(nc, nc, descending=True)
                take = kd < bk
                bk2 = jnp.minimum(bk, kd)
                bv2 = jnp.where(take, vd, bv)
                bn2 = jnp.minimum(bn, nd)
                bk3, bv3 = plsc.sort_key_val(bk2, bv2)
                bn3, _ = plsc.sort_key_val(bn2, bn2)
                return (bk3, bv3, bn3)

            bk, bv, bn = lax.fori_loop(lo_s, hi_s, chunk_body,
                                       (inf16, zeros, inf16))
            accg = accg + jnp.where(m8, bk, zeros)
            accn = accn + jnp.where(m8, bn, zeros)
            accv = accv + jnp.where(m8, bv, zeros)
            return (accg, accn, accv)

        accg, accn, accv = lax.fori_loop(0, _RPW, row_body,
                                         (zeros, zeros, zeros))
        outv[0, :] = accg
        outv[1, :] = accn
        outv[2, :] = accv
        outv[3, :] = zeros
        pltpu.sync_copy(outv, out_hbm.at[wid])

    return body(dg, dn, dv, lo, hi, perm)


def kernel(expected, actual):
    pe = [expected[:, :, :, c] for c in range(3)]
    pa = [actual[:, :, :, c] for c in range(3)]
    ge = jnp.transpose(expected[0, 0, :, 3:])
    ga = jnp.transpose(actual[0, 0, :, 3:])
    parts = []
    for h in range(_B // _HB):
        if h == 0:
            dgh, dnh, dvh, lo, hi, perm = _tc_distances_with_perm(
                pe, pa, ge, ga)
            perm32 = perm[:, 0].reshape(_N // 16, 16)
        else:
            dgh, dnh, dvh = _tc_distances(pe, pa, ge, ga, h)
        parts.append(_sc_topk(dgh.reshape(_HB * _N, _N),
                              dnh.reshape(_HB * _N, _N),
                              dvh.reshape(_HB * _N, _N), lo, hi, perm32))
    tot = jnp.sum(jnp.stack(parts), axis=(0, 1, 3))
    denom = np.float32(_B * _N * _K * math.sqrt(_F))
    return (tot[0] / denom, tot[1] / denom, tot[2] / denom)
